# baseline (speedup 1.0000x reference)
import functools

import jax
import jax.numpy as jnp
from jax import lax
from jax.experimental import pallas as pl
from jax.experimental.pallas import tpu as pltpu
from jax.experimental.pallas import tpu_sc as plsc

E = 8
H = 1024
FF = 2048
S = 2048
NSLOT = 2 * S
R = 256
G = 24
NR = G * R
LANES = 16
NW = 32
GW_PAD = 128

_mesh = plsc.VectorSubcoreMesh(core_axis_name="c", subcore_axis_name="s")


def _router_body(x_ref, gw_ref, e0_ref, e1_ref, w0_ref, w1_ref):
    l = jnp.dot(x_ref[...], gw_ref[...], preferred_element_type=jnp.float32)
    idx = lax.broadcasted_iota(jnp.int32, l.shape, 1)
    l = jnp.where(idx < E, l, -jnp.inf)
    m0 = jnp.max(l, axis=1, keepdims=True)
    e0 = jnp.min(jnp.where(l == m0, idx, E), axis=1, keepdims=True)
    l2 = jnp.where(idx == e0, -jnp.inf, l)
    m1 = jnp.max(l2, axis=1, keepdims=True)
    e1 = jnp.min(jnp.where(l2 == m1, idx, E), axis=1, keepdims=True)
    p0 = 1.0 / (1.0 + jnp.exp(m1 - m0))
    e0_ref[...] = e0
    e1_ref[...] = e1
    w0_ref[...] = p0
    w1_ref[...] = 1.0 - p0


_router = pl.pallas_call(
    _router_body,
    out_shape=[
        jax.ShapeDtypeStruct((S, 1), jnp.int32),
        jax.ShapeDtypeStruct((S, 1), jnp.int32),
        jax.ShapeDtypeStruct((S, 1), jnp.float32),
        jax.ShapeDtypeStruct((S, 1), jnp.float32),
    ],
)


@functools.partial(
    pl.kernel,
    mesh=_mesh,
    compiler_params=pltpu.CompilerParams(needs_layout_passes=False),
    out_type=[
        jax.ShapeDtypeStruct((32,), jnp.int32),
        jax.ShapeDtypeStruct((NR,), jnp.int32),
        jax.ShapeDtypeStruct((NR,), jnp.float32),
        jax.ShapeDtypeStruct((NSLOT,), jnp.int32),
    ],
    scratch_types=[
        pltpu.VMEM((NSLOT,), jnp.int32),
        pltpu.VMEM((NSLOT,), jnp.float32),
        pltpu.VMEM((NR,), jnp.int32),
        pltpu.VMEM((NR,), jnp.float32),
        pltpu.VMEM((NSLOT,), jnp.int32),
        pltpu.VMEM((32,), jnp.int32),
    ],
)
def _dispatch_meta(e0_hbm, e1_hbm, w0_hbm, w1_hbm,
                   be_hbm, rt_hbm, rw_hbm, pos_hbm,
                   sel_v, w_v, rt_v, rw_v, pos_v, be_v):
    wid = lax.axis_index("s") * 2 + lax.axis_index("c")

    @pl.when(wid == 0)
    def _():
        pltpu.sync_copy(e0_hbm, sel_v.at[pl.ds(0, S)])
        pltpu.sync_copy(e1_hbm, sel_v.at[pl.ds(S, S)])
        pltpu.sync_copy(w0_hbm, w_v.at[pl.ds(0, S)])
        pltpu.sync_copy(w1_hbm, w_v.at[pl.ds(S, S)])

        nvec = NSLOT // LANES

        def hist_body(i, acc):
            v = sel_v[pl.ds(i * LANES, LANES)]
            return tuple(acc[e] + (v == e).astype(jnp.int32) for e in range(E))

        acc = lax.fori_loop(
            0, nvec, hist_body,
            tuple(jnp.zeros((LANES,), jnp.int32) for _ in range(E)))
        tot = [jnp.sum(a) for a in acc]
        nb = [(t + (R - 1)) >> 8 for t in tot]
        bs = []
        run = jnp.int32(0)
        for e in range(E):
            bs.append(run)
            run = run + nb[e]
        tb = run

        def z_body(i, _):
            rt_v[pl.ds(i * LANES, LANES)] = jnp.zeros((LANES,), jnp.int32)
            rw_v[pl.ds(i * LANES, LANES)] = jnp.zeros((LANES,), jnp.float32)
            return 0

        lax.fori_loop(0, NR // LANES, z_body, 0)

        def p2_body(i, cur):
            v = sel_v[pl.ds(i * LANES, LANES)]
            w = w_v[pl.ds(i * LANES, LANES)]
            pos = jnp.zeros((LANES,), jnp.int32)
            ncur = []
            for e in range(E):
                m = v == e
                mi = m.astype(jnp.int32)
                csum = plsc.cumsum(mi)
                pos = jnp.where(m, cur[e] + csum - 1, pos)
                ncur.append(cur[e] + jnp.sum(mi))
            tok = (i * LANES + lax.iota(jnp.int32, 16)) & (S - 1)
            plsc.store_scatter(rt_v, [pos], tok)
            plsc.store_scatter(rw_v, [pos], w)
            pos_v[pl.ds(i * LANES, LANES)] = pos
            return tuple(ncur)

        lax.fori_loop(0, nvec, p2_body, tuple(bs[e] * R for e in range(E)))

        for j in range(2):
            g = lax.iota(jnp.int32, 16) + j * LANES
            ge = jnp.minimum(g, tb - 1)
            be = jnp.zeros((LANES,), jnp.int32)
            for e in range(1, E):
                be = be + (ge >= bs[e]).astype(jnp.int32)
            be_v[pl.ds(j * LANES, LANES)] = be

        pltpu.sync_copy(be_v, be_hbm)
        pltpu.sync_copy(rt_v, rt_hbm)
        pltpu.sync_copy(rw_v, rw_hbm)
        pltpu.sync_copy(pos_v, pos_hbm)


_ROWS_PER = NR // NW
_GCH = 48
_GNB = 2
_GNCH = _ROWS_PER // _GCH


@functools.partial(
    pl.kernel,
    mesh=_mesh,
    compiler_params=pltpu.CompilerParams(needs_layout_passes=False),
    out_type=jax.ShapeDtypeStruct((2 * NR, H // 2), jnp.float32),
    scratch_types=[
        pltpu.VMEM((_ROWS_PER,), jnp.int32),
        pltpu.VMEM((2 * _ROWS_PER,), jnp.int32),
    ] + [pltpu.VMEM((2 * _GCH, H // 2), jnp.float32) for _ in range(_GNB)]
      + [pltpu.SemaphoreType.DMA for _ in range(2 * _GNB)],
)
def _gather_rows(rt_hbm, x_hbm, ex_hbm, idx_v, idx2_v, *bufsem):
    bufs = bufsem[:_GNB]
    gs = bufsem[_GNB:2 * _GNB]
    ss = bufsem[2 * _GNB:]
    wid = lax.axis_index("s") * 2 + lax.axis_index("c")
    base = wid * _ROWS_PER
    pltpu.sync_copy(rt_hbm.at[pl.ds(base, _ROWS_PER)], idx_v)

    def dbl(i, _):
        v = idx_v[pl.ds(i * LANES, LANES)]
        idx2_v[pl.ds(2 * i * LANES, LANES)] = v * 2
        idx2_v[pl.ds((2 * i + 1) * LANES, LANES)] = v * 2 + 1
        return 0

    lax.fori_loop(0, _ROWS_PER // LANES, dbl, 0)

    g_desc = [None] * _GNCH
    s_desc = [None] * _GNCH
    for c in range(_GNB):
        g_desc[c] = pltpu.async_copy(
            x_hbm.at[idx2_v.at[pl.ds(c * 2 * _GCH, 2 * _GCH)]], bufs[c], gs[c])
    for c in range(_GNCH):
        b = c % _GNB
        g_desc[c].wait()
        s_desc[c] = pltpu.async_copy(
            bufs[b], ex_hbm.at[pl.ds(2 * (base + c * _GCH), 2 * _GCH)], ss[b])
        nc = c + _GNB
        if nc < _GNCH:
            s_desc[c].wait()
            g_desc[nc] = pltpu.async_copy(
                x_hbm.at[idx2_v.at[pl.ds(nc * 2 * _GCH, 2 * _GCH)]], bufs[b], gs[b])
    for c in range(_GNCH - _GNB, _GNCH):
        s_desc[c].wait()


def _mlp_body(be_ref, ex_ref, w1_ref, w3_ref, w2_ref, rw_ref, y_ref):
    x = ex_ref[...]
    a = jnp.dot(x, w1_ref[0], preferred_element_type=jnp.float32)
    u = jnp.dot(x, w3_ref[0], preferred_element_type=jnp.float32)
    h = (a / (1.0 + jnp.exp(-a))) * u
    y = jnp.dot(h, w2_ref[0], preferred_element_type=jnp.float32)
    y_ref[...] = y * rw_ref[...]


_mlp = pl.pallas_call(
    _mlp_body,
    grid_spec=pltpu.PrefetchScalarGridSpec(
        num_scalar_prefetch=1,
        grid=(G,),
        in_specs=[
            pl.BlockSpec((R, H), lambda g, be: (g, 0)),
            pl.BlockSpec((1, H, FF), lambda g, be: (be[g], 0, 0)),
            pl.BlockSpec((1, H, FF), lambda g, be: (be[g], 0, 0)),
            pl.BlockSpec((1, FF, H), lambda g, be: (be[g], 0, 0)),
            pl.BlockSpec((R, 1), lambda g, be: (g, 0)),
        ],
        out_specs=pl.BlockSpec((R, H), lambda g, be: (g, 0)),
    ),
    out_shape=jax.ShapeDtypeStruct((NR, H), jnp.float32),
    compiler_params=pltpu.CompilerParams(
        dimension_semantics=("arbitrary",),
        vmem_limit_bytes=100 * 1024 * 1024,
    ),
)


_TOK_PER = S // NW
_MCH = 16


_MNCH = _TOK_PER // _MCH


@functools.partial(
    pl.kernel,
    mesh=_mesh,
    compiler_params=pltpu.CompilerParams(needs_layout_passes=False),
    out_type=jax.ShapeDtypeStruct((S, H), jnp.float32),
    scratch_types=[
        pltpu.VMEM((_TOK_PER,), jnp.int32),
        pltpu.VMEM((_TOK_PER,), jnp.int32),
    ] + [pltpu.VMEM((_MCH, H), jnp.float32) for _ in range(4)]
      + [pltpu.SemaphoreType.DMA for _ in range(6)],
)
def _merge(y_hbm, pos_hbm, out_hbm, p0_v, p1_v, *bufsem):
    y0 = bufsem[0:2]
    y1 = bufsem[2:4]
    sa = bufsem[4:6]
    sb = bufsem[6:8]
    so = bufsem[8:10]
    wid = lax.axis_index("s") * 2 + lax.axis_index("c")
    base = wid * _TOK_PER
    pltpu.sync_copy(pos_hbm.at[pl.ds(base, _TOK_PER)], p0_v)
    pltpu.sync_copy(pos_hbm.at[pl.ds(S + base, _TOK_PER)], p1_v)

    d0 = [None] * _MNCH
    d1 = [None] * _MNCH
    for c in range(2):
        d0[c] = pltpu.async_copy(
            y_hbm.at[p0_v.at[pl.ds(c * _MCH, _MCH)]], y0[c % 2], sa[c % 2])
        d1[c] = pltpu.async_copy(
            y_hbm.at[p1_v.at[pl.ds(c * _MCH, _MCH)]], y1[c % 2], sb[c % 2])
    for c in range(_MNCH):
        p = c % 2
        d0[c].wait()
        d1[c].wait()

        def add_body(i, _, y0r=y0[p], y1r=y1[p]):
            r = i >> 6
            col = (i & 63) * LANES
            y0r[r, pl.ds(col, LANES)] = (
                y0r[r, pl.ds(col, LANES)] + y1r[r, pl.ds(col, LANES)])
            return 0

        lax.fori_loop(0, _MCH * (H // LANES), add_body, 0)
        od = pltpu.async_copy(
            y0[p], out_hbm.at[pl.ds(base + c * _MCH, _MCH)], so[p])
        if c + 2 < _MNCH:
            od.wait()
            d0[c + 2] = pltpu.async_copy(
                y_hbm.at[p0_v.at[pl.ds((c + 2) * _MCH, _MCH)]], y0[p], sa[p])
            d1[c + 2] = pltpu.async_copy(
                y_hbm.at[p1_v.at[pl.ds((c + 2) * _MCH, _MCH)]], y1[p], sb[p])
        else:
            od.wait()


def kernel(hidden_states, gate_w, w1, w2, w3):
    b, s, h = hidden_states.shape
    x2d = hidden_states.reshape(s, h)
    gw_pad = jnp.pad(gate_w, ((0, 0), (0, GW_PAD - E)))
    e0, e1, wt0, wt1 = _router(x2d, gw_pad)
    be, rt, rw, pos = _dispatch_meta(
        e0.reshape(S), e1.reshape(S), wt0.reshape(S), wt1.reshape(S))
    ex = _gather_rows(rt, x2d.reshape(2 * S, H // 2))
    py = _mlp(be, ex.reshape(NR, H), w1, w3, w2, rw.reshape(NR, 1))
    out = _merge(py, pos)
    return out.reshape(b, s, h)


# speedup vs baseline: 1.5499x; 1.5499x over previous
import functools

import jax
import jax.numpy as jnp
from jax import lax
from jax.experimental import pallas as pl
from jax.experimental.pallas import tpu as pltpu
from jax.experimental.pallas import tpu_sc as plsc

E = 8
H = 1024
FF = 2048
S = 2048
NSLOT = 2 * S
R = 256
G = 24
NR = G * R
LANES = 16
NW = 32
GW_PAD = 128

_mesh = plsc.VectorSubcoreMesh(core_axis_name="c", subcore_axis_name="s")


def _router_body(x_ref, gw_ref, e0_ref, e1_ref, w0_ref, w1_ref):
    l = jnp.dot(x_ref[...], gw_ref[...], preferred_element_type=jnp.float32)
    idx = lax.broadcasted_iota(jnp.int32, l.shape, 1)
    l = jnp.where(idx < E, l, -jnp.inf)
    m0 = jnp.max(l, axis=1, keepdims=True)
    e0 = jnp.min(jnp.where(l == m0, idx, E), axis=1, keepdims=True)
    l2 = jnp.where(idx == e0, -jnp.inf, l)
    m1 = jnp.max(l2, axis=1, keepdims=True)
    e1 = jnp.min(jnp.where(l2 == m1, idx, E), axis=1, keepdims=True)
    p0 = 1.0 / (1.0 + jnp.exp(m1 - m0))
    e0_ref[...] = e0
    e1_ref[...] = e1
    w0_ref[...] = p0
    w1_ref[...] = 1.0 - p0


_router = pl.pallas_call(
    _router_body,
    out_shape=[
        jax.ShapeDtypeStruct((S, 1), jnp.int32),
        jax.ShapeDtypeStruct((S, 1), jnp.int32),
        jax.ShapeDtypeStruct((S, 1), jnp.float32),
        jax.ShapeDtypeStruct((S, 1), jnp.float32),
    ],
)


@functools.partial(
    pl.kernel,
    mesh=_mesh,
    compiler_params=pltpu.CompilerParams(needs_layout_passes=False),
    out_type=[
        jax.ShapeDtypeStruct((32,), jnp.int32),
        jax.ShapeDtypeStruct((NR,), jnp.int32),
        jax.ShapeDtypeStruct((NR,), jnp.float32),
        jax.ShapeDtypeStruct((NSLOT,), jnp.int32),
    ],
    scratch_types=[
        pltpu.VMEM((NSLOT,), jnp.int32),
        pltpu.VMEM((NSLOT,), jnp.float32),
        pltpu.VMEM((NR,), jnp.int32),
        pltpu.VMEM((NR,), jnp.float32),
        pltpu.VMEM((NSLOT,), jnp.int32),
        pltpu.VMEM((32,), jnp.int32),
    ],
)
def _dispatch_meta(e0_hbm, e1_hbm, w0_hbm, w1_hbm,
                   be_hbm, rt_hbm, rw_hbm, pos_hbm,
                   sel_v, w_v, rt_v, rw_v, pos_v, be_v):
    wid = lax.axis_index("s") * 2 + lax.axis_index("c")

    @pl.when(wid == 0)
    def _():
        pltpu.sync_copy(e0_hbm, sel_v.at[pl.ds(0, S)])
        pltpu.sync_copy(e1_hbm, sel_v.at[pl.ds(S, S)])
        pltpu.sync_copy(w0_hbm, w_v.at[pl.ds(0, S)])
        pltpu.sync_copy(w1_hbm, w_v.at[pl.ds(S, S)])

        nvec = NSLOT // LANES

        def hist_body(i, acc):
            v = sel_v[pl.ds(i * LANES, LANES)]
            return tuple(acc[e] + (v == e).astype(jnp.int32) for e in range(E))

        acc = lax.fori_loop(
            0, nvec, hist_body,
            tuple(jnp.zeros((LANES,), jnp.int32) for _ in range(E)))
        tot = [jnp.sum(a) for a in acc]
        nb = [(t + (R - 1)) >> 8 for t in tot]
        bs = []
        run = jnp.int32(0)
        for e in range(E):
            bs.append(run)
            run = run + nb[e]
        tb = run

        def z_body(i, _):
            rt_v[pl.ds(i * LANES, LANES)] = jnp.zeros((LANES,), jnp.int32)
            rw_v[pl.ds(i * LANES, LANES)] = jnp.zeros((LANES,), jnp.float32)
            return 0

        lax.fori_loop(0, NR // LANES, z_body, 0)

        def p2_body(i, cur):
            v = sel_v[pl.ds(i * LANES, LANES)]
            w = w_v[pl.ds(i * LANES, LANES)]
            pos = jnp.zeros((LANES,), jnp.int32)
            ncur = []
            for e in range(E):
                m = v == e
                mi = m.astype(jnp.int32)
                csum = plsc.cumsum(mi)
                pos = jnp.where(m, cur[e] + csum - 1, pos)
                ncur.append(cur[e] + jnp.sum(mi))
            tok = (i * LANES + lax.iota(jnp.int32, 16)) & (S - 1)
            plsc.store_scatter(rt_v, [pos], tok)
            plsc.store_scatter(rw_v, [pos], w)
            pos_v[pl.ds(i * LANES, LANES)] = pos
            return tuple(ncur)

        lax.fori_loop(0, nvec, p2_body, tuple(bs[e] * R for e in range(E)))

        for j in range(2):
            g = lax.iota(jnp.int32, 16) + j * LANES
            ge = jnp.minimum(g, tb - 1)
            be = jnp.zeros((LANES,), jnp.int32)
            for e in range(1, E):
                be = be + (ge >= bs[e]).astype(jnp.int32)
            be_v[pl.ds(j * LANES, LANES)] = be

        pltpu.sync_copy(be_v, be_hbm)
        pltpu.sync_copy(rt_v, rt_hbm)
        pltpu.sync_copy(rw_v, rw_hbm)
        pltpu.sync_copy(pos_v, pos_hbm)


def _mlp_body(be_ref, rt_ref, x_ref, w1_ref, w3_ref, w2_ref, rw_ref, y_ref):
    tok = rt_ref[0]
    ids = lax.broadcasted_iota(jnp.int32, (R, S), 1)
    onehot = (ids == tok).astype(jnp.float32)
    x = jnp.dot(onehot, x_ref[...], preferred_element_type=jnp.float32)
    a = jnp.dot(x, w1_ref[0], preferred_element_type=jnp.float32)
    u = jnp.dot(x, w3_ref[0], preferred_element_type=jnp.float32)
    h = (a / (1.0 + jnp.exp(-a))) * u
    y = jnp.dot(h, w2_ref[0], preferred_element_type=jnp.float32)
    y_ref[...] = y * rw_ref[...]


_mlp = pl.pallas_call(
    _mlp_body,
    grid_spec=pltpu.PrefetchScalarGridSpec(
        num_scalar_prefetch=1,
        grid=(G,),
        in_specs=[
            pl.BlockSpec((1, R, 1), lambda g, be: (g, 0, 0)),
            pl.BlockSpec((S, H), lambda g, be: (0, 0)),
            pl.BlockSpec((1, H, FF), lambda g, be: (be[g], 0, 0)),
            pl.BlockSpec((1, H, FF), lambda g, be: (be[g], 0, 0)),
            pl.BlockSpec((1, FF, H), lambda g, be: (be[g], 0, 0)),
            pl.BlockSpec((R, 1), lambda g, be: (g, 0)),
        ],
        out_specs=pl.BlockSpec((R, H), lambda g, be: (g, 0)),
    ),
    out_shape=jax.ShapeDtypeStruct((NR, H), jnp.float32),
    compiler_params=pltpu.CompilerParams(
        dimension_semantics=("arbitrary",),
        vmem_limit_bytes=100 * 1024 * 1024,
    ),
)


_TOK_PER = S // NW
_MCH = 16


_MNCH = _TOK_PER // _MCH


@functools.partial(
    pl.kernel,
    mesh=_mesh,
    compiler_params=pltpu.CompilerParams(needs_layout_passes=False),
    out_type=jax.ShapeDtypeStruct((S, H), jnp.float32),
    scratch_types=[
        pltpu.VMEM((_TOK_PER,), jnp.int32),
        pltpu.VMEM((_TOK_PER,), jnp.int32),
    ] + [pltpu.VMEM((_MCH, H), jnp.float32) for _ in range(4)]
      + [pltpu.SemaphoreType.DMA for _ in range(6)],
)
def _merge(y_hbm, pos_hbm, out_hbm, p0_v, p1_v, *bufsem):
    y0 = bufsem[0:2]
    y1 = bufsem[2:4]
    sa = bufsem[4:6]
    sb = bufsem[6:8]
    so = bufsem[8:10]
    wid = lax.axis_index("s") * 2 + lax.axis_index("c")
    base = wid * _TOK_PER
    pltpu.sync_copy(pos_hbm.at[pl.ds(base, _TOK_PER)], p0_v)
    pltpu.sync_copy(pos_hbm.at[pl.ds(S + base, _TOK_PER)], p1_v)

    d0 = [None] * _MNCH
    d1 = [None] * _MNCH
    for c in range(2):
        d0[c] = pltpu.async_copy(
            y_hbm.at[p0_v.at[pl.ds(c * _MCH, _MCH)]], y0[c % 2], sa[c % 2])
        d1[c] = pltpu.async_copy(
            y_hbm.at[p1_v.at[pl.ds(c * _MCH, _MCH)]], y1[c % 2], sb[c % 2])
    for c in range(_MNCH):
        p = c % 2
        d0[c].wait()
        d1[c].wait()

        def add_body(i, _, y0r=y0[p], y1r=y1[p]):
            r = i >> 6
            col = (i & 63) * LANES
            y0r[r, pl.ds(col, LANES)] = (
                y0r[r, pl.ds(col, LANES)] + y1r[r, pl.ds(col, LANES)])
            return 0

        lax.fori_loop(0, _MCH * (H // LANES), add_body, 0)
        od = pltpu.async_copy(
            y0[p], out_hbm.at[pl.ds(base + c * _MCH, _MCH)], so[p])
        if c + 2 < _MNCH:
            od.wait()
            d0[c + 2] = pltpu.async_copy(
                y_hbm.at[p0_v.at[pl.ds((c + 2) * _MCH, _MCH)]], y0[p], sa[p])
            d1[c + 2] = pltpu.async_copy(
                y_hbm.at[p1_v.at[pl.ds((c + 2) * _MCH, _MCH)]], y1[p], sb[p])
        else:
            od.wait()


def kernel(hidden_states, gate_w, w1, w2, w3):
    b, s, h = hidden_states.shape
    x2d = hidden_states.reshape(s, h)
    gw_pad = jnp.pad(gate_w, ((0, 0), (0, GW_PAD - E)))
    e0, e1, wt0, wt1 = _router(x2d, gw_pad)
    be, rt, rw, pos = _dispatch_meta(
        e0.reshape(S), e1.reshape(S), wt0.reshape(S), wt1.reshape(S))
    py = _mlp(be, rt.reshape(G, R, 1), x2d, w1, w3, w2, rw.reshape(NR, 1))
    out = _merge(py, pos)
    return out.reshape(b, s, h)


# speedup vs baseline: 1.6598x; 1.0709x over previous
import functools

import jax
import jax.numpy as jnp
from jax import lax
from jax.experimental import pallas as pl
from jax.experimental.pallas import tpu as pltpu
from jax.experimental.pallas import tpu_sc as plsc

E = 8
H = 1024
FF = 2048
S = 2048
NSLOT = 2 * S
R = 256
G = 24
NR = G * R
LANES = 16
NW = 32
GW_PAD = 128

_mesh = plsc.VectorSubcoreMesh(core_axis_name="c", subcore_axis_name="s")


def _router_body(x_ref, gw_ref, e0_ref, e1_ref, w0_ref, w1_ref):
    l = jnp.dot(x_ref[...], gw_ref[...], preferred_element_type=jnp.float32)
    idx = lax.broadcasted_iota(jnp.int32, l.shape, 1)
    m0 = jnp.max(l, axis=1, keepdims=True)
    e0 = jnp.min(jnp.where(l == m0, idx, E), axis=1, keepdims=True)
    l2 = jnp.where(idx == e0, -jnp.inf, l)
    m1 = jnp.max(l2, axis=1, keepdims=True)
    e1 = jnp.min(jnp.where(l2 == m1, idx, E), axis=1, keepdims=True)
    p0 = 1.0 / (1.0 + jnp.exp(m1 - m0))
    e0_ref[...] = e0
    e1_ref[...] = e1
    w0_ref[...] = p0
    w1_ref[...] = 1.0 - p0


_router = pl.pallas_call(
    _router_body,
    out_shape=[
        jax.ShapeDtypeStruct((S, 1), jnp.int32),
        jax.ShapeDtypeStruct((S, 1), jnp.int32),
        jax.ShapeDtypeStruct((S, 1), jnp.float32),
        jax.ShapeDtypeStruct((S, 1), jnp.float32),
    ],
)


@functools.partial(
    pl.kernel,
    mesh=_mesh,
    compiler_params=pltpu.CompilerParams(needs_layout_passes=False),
    out_type=[
        jax.ShapeDtypeStruct((32,), jnp.int32),
        jax.ShapeDtypeStruct((NR,), jnp.int32),
        jax.ShapeDtypeStruct((NR,), jnp.float32),
        jax.ShapeDtypeStruct((NSLOT,), jnp.int32),
    ],
    scratch_types=[
        pltpu.VMEM((NSLOT,), jnp.int32),
        pltpu.VMEM((NSLOT,), jnp.float32),
        pltpu.VMEM((NR,), jnp.int32),
        pltpu.VMEM((NR,), jnp.float32),
        pltpu.VMEM((NSLOT,), jnp.int32),
        pltpu.VMEM((32,), jnp.int32),
    ],
)
def _dispatch_meta(e0_hbm, e1_hbm, w0_hbm, w1_hbm,
                   be_hbm, rt_hbm, rw_hbm, pos_hbm,
                   sel_v, w_v, rt_v, rw_v, pos_v, be_v):
    wid = lax.axis_index("s") * 2 + lax.axis_index("c")

    @pl.when(wid == 0)
    def _():
        pltpu.sync_copy(e0_hbm, sel_v.at[pl.ds(0, S)])
        pltpu.sync_copy(e1_hbm, sel_v.at[pl.ds(S, S)])
        pltpu.sync_copy(w0_hbm, w_v.at[pl.ds(0, S)])
        pltpu.sync_copy(w1_hbm, w_v.at[pl.ds(S, S)])

        nvec = NSLOT // LANES

        def hist_body(i, acc):
            v = sel_v[pl.ds(i * LANES, LANES)]
            return tuple(acc[e] + (v == e).astype(jnp.int32) for e in range(E))

        acc = lax.fori_loop(
            0, nvec, hist_body,
            tuple(jnp.zeros((LANES,), jnp.int32) for _ in range(E)))
        tot = [jnp.sum(a) for a in acc]
        nb = [(t + (R - 1)) >> 8 for t in tot]
        bs = []
        run = jnp.int32(0)
        for e in range(E):
            bs.append(run)
            run = run + nb[e]
        tb = run

        def z_body(i, _):
            rt_v[pl.ds(i * LANES, LANES)] = jnp.zeros((LANES,), jnp.int32)
            rw_v[pl.ds(i * LANES, LANES)] = jnp.zeros((LANES,), jnp.float32)
            return 0

        lax.fori_loop(0, NR // LANES, z_body, 0)

        def p2_body(i, cur):
            v = sel_v[pl.ds(i * LANES, LANES)]
            w = w_v[pl.ds(i * LANES, LANES)]
            pos = jnp.zeros((LANES,), jnp.int32)
            ncur = []
            for e in range(E):
                m = v == e
                mi = m.astype(jnp.int32)
                csum = plsc.cumsum(mi)
                pos = jnp.where(m, cur[e] + csum - 1, pos)
                ncur.append(cur[e] + jnp.sum(mi))
            tok = (i * LANES + lax.iota(jnp.int32, 16)) & (S - 1)
            plsc.store_scatter(rt_v, [pos], tok)
            plsc.store_scatter(rw_v, [pos], w)
            pos_v[pl.ds(i * LANES, LANES)] = pos
            return tuple(ncur)

        lax.fori_loop(0, nvec, p2_body, tuple(bs[e] * R for e in range(E)))

        for j in range(2):
            g = lax.iota(jnp.int32, 16) + j * LANES
            ge = jnp.minimum(g, tb - 1)
            be = jnp.zeros((LANES,), jnp.int32)
            for e in range(1, E):
                be = be + (ge >= bs[e]).astype(jnp.int32)
            if j == 1:
                be = jnp.where(lax.iota(jnp.int32, 16) == 15, tb, be)
            be_v[pl.ds(j * LANES, LANES)] = be

        pltpu.sync_copy(be_v, be_hbm)
        pltpu.sync_copy(rt_v, rt_hbm)
        pltpu.sync_copy(rw_v, rw_hbm)
        pltpu.sync_copy(pos_v, pos_hbm)


def _mlp_body(be_ref, rt_ref, x_ref, w1_ref, w3_ref, w2_ref, rw_ref, y_ref):
    g = pl.program_id(0)

    @pl.when(g < be_ref[31])
    def _():
        tok = rt_ref[0]
        ids = lax.broadcasted_iota(jnp.int32, (R, S), 1)
        onehot = (ids == tok).astype(jnp.float32)
        x = jnp.dot(onehot, x_ref[...], preferred_element_type=jnp.float32)
        a = jnp.dot(x, w1_ref[0], preferred_element_type=jnp.float32)
        u = jnp.dot(x, w3_ref[0], preferred_element_type=jnp.float32)
        h = (a / (1.0 + jnp.exp(-a))) * u
        y = jnp.dot(h, w2_ref[0], preferred_element_type=jnp.float32)
        y_ref[...] = y * rw_ref[...]


_mlp = pl.pallas_call(
    _mlp_body,
    grid_spec=pltpu.PrefetchScalarGridSpec(
        num_scalar_prefetch=1,
        grid=(G,),
        in_specs=[
            pl.BlockSpec((1, R, 1), lambda g, be: (g, 0, 0)),
            pl.BlockSpec((S, H), lambda g, be: (0, 0)),
            pl.BlockSpec((1, H, FF), lambda g, be: (be[g], 0, 0)),
            pl.BlockSpec((1, H, FF), lambda g, be: (be[g], 0, 0)),
            pl.BlockSpec((1, FF, H), lambda g, be: (be[g], 0, 0)),
            pl.BlockSpec((R, 1), lambda g, be: (g, 0)),
        ],
        out_specs=pl.BlockSpec((R, H), lambda g, be: (g, 0)),
    ),
    out_shape=jax.ShapeDtypeStruct((NR, H), jnp.float32),
    compiler_params=pltpu.CompilerParams(
        dimension_semantics=("arbitrary",),
        vmem_limit_bytes=100 * 1024 * 1024,
    ),
)


_TOK_PER = S // NW
_MCH = 16


_MNCH = _TOK_PER // _MCH


@functools.partial(
    pl.kernel,
    mesh=_mesh,
    compiler_params=pltpu.CompilerParams(needs_layout_passes=False),
    out_type=jax.ShapeDtypeStruct((S, H), jnp.float32),
    scratch_types=[
        pltpu.VMEM((_TOK_PER,), jnp.int32),
        pltpu.VMEM((_TOK_PER,), jnp.int32),
    ] + [pltpu.VMEM((_MCH, H), jnp.float32) for _ in range(4)]
      + [pltpu.SemaphoreType.DMA for _ in range(6)],
)
def _merge(y_hbm, pos_hbm, out_hbm, p0_v, p1_v, *bufsem):
    y0 = bufsem[0:2]
    y1 = bufsem[2:4]
    sa = bufsem[4:6]
    sb = bufsem[6:8]
    so = bufsem[8:10]
    wid = lax.axis_index("s") * 2 + lax.axis_index("c")
    base = wid * _TOK_PER
    pltpu.sync_copy(pos_hbm.at[pl.ds(base, _TOK_PER)], p0_v)
    pltpu.sync_copy(pos_hbm.at[pl.ds(S + base, _TOK_PER)], p1_v)

    d0 = [None] * _MNCH
    d1 = [None] * _MNCH
    for c in range(2):
        d0[c] = pltpu.async_copy(
            y_hbm.at[p0_v.at[pl.ds(c * _MCH, _MCH)]], y0[c % 2], sa[c % 2])
        d1[c] = pltpu.async_copy(
            y_hbm.at[p1_v.at[pl.ds(c * _MCH, _MCH)]], y1[c % 2], sb[c % 2])
    for c in range(_MNCH):
        p = c % 2
        d0[c].wait()
        d1[c].wait()

        def add_body(i, _, y0r=y0[p], y1r=y1[p]):
            r = i >> 6
            col = (i & 63) * LANES
            y0r[r, pl.ds(col, LANES)] = (
                y0r[r, pl.ds(col, LANES)] + y1r[r, pl.ds(col, LANES)])
            return 0

        lax.fori_loop(0, _MCH * (H // LANES), add_body, 0)
        od = pltpu.async_copy(
            y0[p], out_hbm.at[pl.ds(base + c * _MCH, _MCH)], so[p])
        if c + 2 < _MNCH:
            od.wait()
            d0[c + 2] = pltpu.async_copy(
                y_hbm.at[p0_v.at[pl.ds((c + 2) * _MCH, _MCH)]], y0[p], sa[p])
            d1[c + 2] = pltpu.async_copy(
                y_hbm.at[p1_v.at[pl.ds((c + 2) * _MCH, _MCH)]], y1[p], sb[p])
        else:
            od.wait()


def kernel(hidden_states, gate_w, w1, w2, w3):
    b, s, h = hidden_states.shape
    x2d = hidden_states.reshape(s, h)
    e0, e1, wt0, wt1 = _router(x2d, gate_w)
    be, rt, rw, pos = _dispatch_meta(
        e0.reshape(S), e1.reshape(S), wt0.reshape(S), wt1.reshape(S))
    py = _mlp(be, rt.reshape(G, R, 1), x2d, w1, w3, w2, rw.reshape(NR, 1))
    out = _merge(py, pos)
    return out.reshape(b, s, h)


# speedup vs baseline: 1.7098x; 1.0301x over previous
import functools

import jax
import jax.numpy as jnp
from jax import lax
from jax.experimental import pallas as pl
from jax.experimental.pallas import tpu as pltpu
from jax.experimental.pallas import tpu_sc as plsc

E = 8
H = 1024
FF = 2048
S = 2048
NSLOT = 2 * S
R = 256
G = 24
NR = G * R
LANES = 16
NW = 32
GW_PAD = 128

_mesh = plsc.VectorSubcoreMesh(core_axis_name="c", subcore_axis_name="s")


def _router_body(x_ref, gw_ref, e0_ref, e1_ref, w0_ref, w1_ref):
    l = jnp.dot(x_ref[...], gw_ref[...], preferred_element_type=jnp.float32)
    idx = lax.broadcasted_iota(jnp.int32, l.shape, 1)
    m0 = jnp.max(l, axis=1, keepdims=True)
    e0 = jnp.min(jnp.where(l == m0, idx, E), axis=1, keepdims=True)
    l2 = jnp.where(idx == e0, -jnp.inf, l)
    m1 = jnp.max(l2, axis=1, keepdims=True)
    e1 = jnp.min(jnp.where(l2 == m1, idx, E), axis=1, keepdims=True)
    p0 = 1.0 / (1.0 + jnp.exp(m1 - m0))
    e0_ref[...] = e0.reshape(S)
    e1_ref[...] = e1.reshape(S)
    w0_ref[...] = p0.reshape(S)
    w1_ref[...] = 1.0 - p0.reshape(S)


_router = pl.pallas_call(
    _router_body,
    out_shape=[
        jax.ShapeDtypeStruct((S,), jnp.int32),
        jax.ShapeDtypeStruct((S,), jnp.int32),
        jax.ShapeDtypeStruct((S,), jnp.float32),
        jax.ShapeDtypeStruct((S,), jnp.float32),
    ],
)


@functools.partial(
    pl.kernel,
    mesh=_mesh,
    compiler_params=pltpu.CompilerParams(needs_layout_passes=False),
    out_type=[
        jax.ShapeDtypeStruct((32,), jnp.int32),
        jax.ShapeDtypeStruct((NR,), jnp.int32),
        jax.ShapeDtypeStruct((NR,), jnp.float32),
        jax.ShapeDtypeStruct((NSLOT,), jnp.int32),
    ],
    scratch_types=[
        pltpu.VMEM((NSLOT,), jnp.int32),
        pltpu.VMEM((NSLOT,), jnp.float32),
        pltpu.VMEM((NR,), jnp.int32),
        pltpu.VMEM((NR,), jnp.float32),
        pltpu.VMEM((NSLOT,), jnp.int32),
        pltpu.VMEM((32,), jnp.int32),
    ],
)
def _dispatch_meta(e0_hbm, e1_hbm, w0_hbm, w1_hbm,
                   be_hbm, rt_hbm, rw_hbm, pos_hbm,
                   sel_v, w_v, rt_v, rw_v, pos_v, be_v):
    wid = lax.axis_index("s") * 2 + lax.axis_index("c")

    @pl.when(wid == 0)
    def _():
        pltpu.sync_copy(e0_hbm, sel_v.at[pl.ds(0, S)])
        pltpu.sync_copy(e1_hbm, sel_v.at[pl.ds(S, S)])
        pltpu.sync_copy(w0_hbm, w_v.at[pl.ds(0, S)])
        pltpu.sync_copy(w1_hbm, w_v.at[pl.ds(S, S)])

        nvec = NSLOT // LANES

        def hist_body(i, acc):
            v = sel_v[pl.ds(i * LANES, LANES)]
            return tuple(acc[e] + (v == e).astype(jnp.int32) for e in range(E))

        acc = lax.fori_loop(
            0, nvec, hist_body,
            tuple(jnp.zeros((LANES,), jnp.int32) for _ in range(E)))
        tot = [jnp.sum(a) for a in acc]
        nb = [(t + (R - 1)) >> 8 for t in tot]
        bs = []
        run = jnp.int32(0)
        for e in range(E):
            bs.append(run)
            run = run + nb[e]
        tb = run

        def z_body(i, _):
            rt_v[pl.ds(i * LANES, LANES)] = jnp.zeros((LANES,), jnp.int32)
            rw_v[pl.ds(i * LANES, LANES)] = jnp.zeros((LANES,), jnp.float32)
            return 0

        lax.fori_loop(0, NR // LANES, z_body, 0)

        def p2_body(i, cur):
            v = sel_v[pl.ds(i * LANES, LANES)]
            w = w_v[pl.ds(i * LANES, LANES)]
            pos = jnp.zeros((LANES,), jnp.int32)
            ncur = []
            for e in range(E):
                m = v == e
                mi = m.astype(jnp.int32)
                csum = plsc.cumsum(mi)
                pos = jnp.where(m, cur[e] + csum - 1, pos)
                ncur.append(cur[e] + jnp.sum(mi))
            tok = (i * LANES + lax.iota(jnp.int32, 16)) & (S - 1)
            plsc.store_scatter(rt_v, [pos], tok)
            plsc.store_scatter(rw_v, [pos], w)
            pos_v[pl.ds(i * LANES, LANES)] = pos
            return tuple(ncur)

        lax.fori_loop(0, nvec, p2_body, tuple(bs[e] * R for e in range(E)))

        for j in range(2):
            g = lax.iota(jnp.int32, 16) + j * LANES
            ge = jnp.minimum(g, tb - 1)
            be = jnp.zeros((LANES,), jnp.int32)
            for e in range(1, E):
                be = be + (ge >= bs[e]).astype(jnp.int32)
            if j == 1:
                be = jnp.where(lax.iota(jnp.int32, 16) == 15, tb, be)
            be_v[pl.ds(j * LANES, LANES)] = be

        pltpu.sync_copy(be_v, be_hbm)
        pltpu.sync_copy(rt_v, rt_hbm)
        pltpu.sync_copy(rw_v, rw_hbm)
        pltpu.sync_copy(pos_v, pos_hbm)


def _mlp_body(be_ref, rt_ref, x_ref, w1_ref, w3_ref, w2_ref, rw_ref, y_ref):
    g = pl.program_id(0)

    @pl.when(g < be_ref[31])
    def _():
        tok = rt_ref[0]
        ids = lax.broadcasted_iota(jnp.int32, (R, S), 1)
        onehot = (ids == tok).astype(jnp.float32)
        x = jnp.dot(onehot, x_ref[...], preferred_element_type=jnp.float32)
        a = jnp.dot(x, w1_ref[0], preferred_element_type=jnp.float32)
        u = jnp.dot(x, w3_ref[0], preferred_element_type=jnp.float32)
        h = (a / (1.0 + jnp.exp(-a))) * u
        y = jnp.dot(h, w2_ref[0], preferred_element_type=jnp.float32)
        y_ref[...] = y * rw_ref[...]


_mlp = pl.pallas_call(
    _mlp_body,
    grid_spec=pltpu.PrefetchScalarGridSpec(
        num_scalar_prefetch=1,
        grid=(G,),
        in_specs=[
            pl.BlockSpec((1, R, 1), lambda g, be: (g, 0, 0)),
            pl.BlockSpec((S, H), lambda g, be: (0, 0)),
            pl.BlockSpec((1, H, FF), lambda g, be: (be[g], 0, 0)),
            pl.BlockSpec((1, H, FF), lambda g, be: (be[g], 0, 0)),
            pl.BlockSpec((1, FF, H), lambda g, be: (be[g], 0, 0)),
            pl.BlockSpec((R, 1), lambda g, be: (g, 0)),
        ],
        out_specs=pl.BlockSpec((R, H), lambda g, be: (g, 0)),
    ),
    out_shape=jax.ShapeDtypeStruct((NR, H), jnp.float32),
    compiler_params=pltpu.CompilerParams(
        dimension_semantics=("arbitrary",),
        vmem_limit_bytes=100 * 1024 * 1024,
    ),
)


_TOK_PER = S // NW
_MCH = 16


_MNCH = _TOK_PER // _MCH


@functools.partial(
    pl.kernel,
    mesh=_mesh,
    compiler_params=pltpu.CompilerParams(needs_layout_passes=False),
    out_type=jax.ShapeDtypeStruct((S, H), jnp.float32),
    scratch_types=[
        pltpu.VMEM((_TOK_PER,), jnp.int32),
        pltpu.VMEM((_TOK_PER,), jnp.int32),
    ] + [pltpu.VMEM((_MCH, H), jnp.float32) for _ in range(4)]
      + [pltpu.SemaphoreType.DMA for _ in range(6)],
)
def _merge(y_hbm, pos_hbm, out_hbm, p0_v, p1_v, *bufsem):
    y0 = bufsem[0:2]
    y1 = bufsem[2:4]
    sa = bufsem[4:6]
    sb = bufsem[6:8]
    so = bufsem[8:10]
    wid = lax.axis_index("s") * 2 + lax.axis_index("c")
    base = wid * _TOK_PER
    pltpu.sync_copy(pos_hbm.at[pl.ds(base, _TOK_PER)], p0_v)
    pltpu.sync_copy(pos_hbm.at[pl.ds(S + base, _TOK_PER)], p1_v)

    d0 = [None] * _MNCH
    d1 = [None] * _MNCH
    for c in range(2):
        d0[c] = pltpu.async_copy(
            y_hbm.at[p0_v.at[pl.ds(c * _MCH, _MCH)]], y0[c % 2], sa[c % 2])
        d1[c] = pltpu.async_copy(
            y_hbm.at[p1_v.at[pl.ds(c * _MCH, _MCH)]], y1[c % 2], sb[c % 2])
    for c in range(_MNCH):
        p = c % 2
        d0[c].wait()
        d1[c].wait()

        def add_body(i, _, y0r=y0[p], y1r=y1[p]):
            r = i >> 6
            col = (i & 63) * LANES
            y0r[r, pl.ds(col, LANES)] = (
                y0r[r, pl.ds(col, LANES)] + y1r[r, pl.ds(col, LANES)])
            return 0

        lax.fori_loop(0, _MCH * (H // LANES), add_body, 0)
        od = pltpu.async_copy(
            y0[p], out_hbm.at[pl.ds(base + c * _MCH, _MCH)], so[p])
        if c + 2 < _MNCH:
            od.wait()
            d0[c + 2] = pltpu.async_copy(
                y_hbm.at[p0_v.at[pl.ds((c + 2) * _MCH, _MCH)]], y0[p], sa[p])
            d1[c + 2] = pltpu.async_copy(
                y_hbm.at[p1_v.at[pl.ds((c + 2) * _MCH, _MCH)]], y1[p], sb[p])
        else:
            od.wait()


def kernel(hidden_states, gate_w, w1, w2, w3):
    b, s, h = hidden_states.shape
    x2d = hidden_states.reshape(s, h)
    e0, e1, wt0, wt1 = _router(x2d, gate_w)
    be, rt, rw, pos = _dispatch_meta(e0, e1, wt0, wt1)
    py = _mlp(be, rt.reshape(G, R, 1), x2d, w1, w3, w2, rw.reshape(NR, 1))
    out = _merge(py, pos)
    return out.reshape(b, s, h)


# speedup vs baseline: 1.7852x; 1.0441x over previous
import functools

import jax
import jax.numpy as jnp
from jax import lax
from jax.experimental import pallas as pl
from jax.experimental.pallas import tpu as pltpu
from jax.experimental.pallas import tpu_sc as plsc

E = 8
H = 1024
FF = 2048
S = 2048
NSLOT = 2 * S
R = 256
G = 24
NR = G * R
LANES = 16
NW = 32
GW_PAD = 128

_mesh = plsc.VectorSubcoreMesh(core_axis_name="c", subcore_axis_name="s")


def _router_body(x_ref, gw_ref, e0_ref, e1_ref, w0_ref, w1_ref):
    l = jnp.dot(x_ref[...], gw_ref[...], preferred_element_type=jnp.float32)
    idx = lax.broadcasted_iota(jnp.int32, l.shape, 1)
    m0 = jnp.max(l, axis=1, keepdims=True)
    e0 = jnp.min(jnp.where(l == m0, idx, E), axis=1, keepdims=True)
    l2 = jnp.where(idx == e0, -jnp.inf, l)
    m1 = jnp.max(l2, axis=1, keepdims=True)
    e1 = jnp.min(jnp.where(l2 == m1, idx, E), axis=1, keepdims=True)
    p0 = 1.0 / (1.0 + jnp.exp(m1 - m0))
    e0_ref[...] = e0.reshape(S)
    e1_ref[...] = e1.reshape(S)
    w0_ref[...] = p0.reshape(S)
    w1_ref[...] = 1.0 - p0.reshape(S)


_router = pl.pallas_call(
    _router_body,
    out_shape=[
        jax.ShapeDtypeStruct((S,), jnp.int32),
        jax.ShapeDtypeStruct((S,), jnp.int32),
        jax.ShapeDtypeStruct((S,), jnp.float32),
        jax.ShapeDtypeStruct((S,), jnp.float32),
    ],
)


@functools.partial(
    pl.kernel,
    mesh=_mesh,
    compiler_params=pltpu.CompilerParams(needs_layout_passes=False),
    out_type=[
        jax.ShapeDtypeStruct((32,), jnp.int32),
        jax.ShapeDtypeStruct((NR,), jnp.int32),
        jax.ShapeDtypeStruct((NR,), jnp.float32),
        jax.ShapeDtypeStruct((NSLOT,), jnp.int32),
    ],
    scratch_types=[
        pltpu.VMEM((NSLOT,), jnp.int32),
        pltpu.VMEM((NSLOT,), jnp.float32),
        pltpu.VMEM((NR,), jnp.int32),
        pltpu.VMEM((NR,), jnp.float32),
        pltpu.VMEM((NSLOT,), jnp.int32),
        pltpu.VMEM((32,), jnp.int32),
    ],
)
def _dispatch_meta(e0_hbm, e1_hbm, w0_hbm, w1_hbm,
                   be_hbm, rt_hbm, rw_hbm, pos_hbm,
                   sel_v, w_v, rt_v, rw_v, pos_v, be_v):
    wid = lax.axis_index("s") * 2 + lax.axis_index("c")

    @pl.when(wid == 0)
    def _():
        pltpu.sync_copy(e0_hbm, sel_v.at[pl.ds(0, S)])
        pltpu.sync_copy(e1_hbm, sel_v.at[pl.ds(S, S)])
        pltpu.sync_copy(w0_hbm, w_v.at[pl.ds(0, S)])
        pltpu.sync_copy(w1_hbm, w_v.at[pl.ds(S, S)])

        nvec = NSLOT // LANES

        def hist_body(i, acc):
            v = sel_v[pl.ds(i * LANES, LANES)]
            return tuple(acc[e] + (v == e).astype(jnp.int32) for e in range(E))

        acc = lax.fori_loop(
            0, nvec, hist_body,
            tuple(jnp.zeros((LANES,), jnp.int32) for _ in range(E)))
        tot = [jnp.sum(a) for a in acc]
        nb = [(t + (R - 1)) >> 8 for t in tot]
        bs = []
        run = jnp.int32(0)
        for e in range(E):
            bs.append(run)
            run = run + nb[e]
        tb = run

        def z_body(i, _):
            rt_v[pl.ds(i * LANES, LANES)] = jnp.zeros((LANES,), jnp.int32)
            rw_v[pl.ds(i * LANES, LANES)] = jnp.zeros((LANES,), jnp.float32)
            return 0

        lax.fori_loop(0, NR // LANES, z_body, 0)

        def p2_body(i, cur):
            v = sel_v[pl.ds(i * LANES, LANES)]
            w = w_v[pl.ds(i * LANES, LANES)]
            pos = jnp.zeros((LANES,), jnp.int32)
            ncur = []
            for e in range(E):
                m = v == e
                mi = m.astype(jnp.int32)
                csum = plsc.cumsum(mi)
                pos = jnp.where(m, cur[e] + csum - 1, pos)
                ncur.append(cur[e] + jnp.sum(mi))
            tok = (i * LANES + lax.iota(jnp.int32, 16)) & (S - 1)
            plsc.store_scatter(rt_v, [pos], tok)
            plsc.store_scatter(rw_v, [pos], w)
            pos_v[pl.ds(i * LANES, LANES)] = pos
            return tuple(ncur)

        lax.fori_loop(0, nvec, p2_body, tuple(bs[e] * R for e in range(E)))

        for j in range(2):
            g = lax.iota(jnp.int32, 16) + j * LANES
            ge = jnp.minimum(g, tb - 1)
            be = jnp.zeros((LANES,), jnp.int32)
            for e in range(1, E):
                be = be + (ge >= bs[e]).astype(jnp.int32)
            if j == 1:
                be = jnp.where(lax.iota(jnp.int32, 16) == 15, tb, be)
            be_v[pl.ds(j * LANES, LANES)] = be

        pltpu.sync_copy(be_v, be_hbm)
        pltpu.sync_copy(rt_v, rt_hbm)
        pltpu.sync_copy(rw_v, rw_hbm)
        pltpu.sync_copy(pos_v, pos_hbm)


def _mlp_body(be_ref, rt_ref, x_ref, w1_ref, w3_ref, w2_ref, rw_ref, y_ref):
    g = pl.program_id(0)

    @pl.when(g < be_ref[31])
    def _():
        tok = rt_ref[0]
        ids = lax.broadcasted_iota(jnp.int32, (R, S), 1)
        onehot = (ids == tok).astype(jnp.float32)
        x = jnp.dot(onehot, x_ref[...], preferred_element_type=jnp.float32)
        a = jnp.dot(x, w1_ref[0], preferred_element_type=jnp.float32)
        u = jnp.dot(x, w3_ref[0], preferred_element_type=jnp.float32)
        h = (a / (1.0 + jnp.exp(-a))) * u
        y = jnp.dot(h, w2_ref[0], preferred_element_type=jnp.float32)
        y = y * rw_ref[...]
        u1 = jax.lax.bitcast_convert_type(y[:, :H // 2], jnp.int32)
        u2 = jax.lax.bitcast_convert_type(y[:, H // 2:], jnp.int32)
        lo = ((u1 + 0x8000) >> 16) & 0xFFFF
        hi = (u2 + 0x8000) & (-65536)
        y_ref[...] = lo | hi


_mlp = pl.pallas_call(
    _mlp_body,
    grid_spec=pltpu.PrefetchScalarGridSpec(
        num_scalar_prefetch=1,
        grid=(G,),
        in_specs=[
            pl.BlockSpec((1, R, 1), lambda g, be: (g, 0, 0)),
            pl.BlockSpec((S, H), lambda g, be: (0, 0)),
            pl.BlockSpec((1, H, FF), lambda g, be: (be[g], 0, 0)),
            pl.BlockSpec((1, H, FF), lambda g, be: (be[g], 0, 0)),
            pl.BlockSpec((1, FF, H), lambda g, be: (be[g], 0, 0)),
            pl.BlockSpec((R, 1), lambda g, be: (g, 0)),
        ],
        out_specs=pl.BlockSpec((R, H // 2), lambda g, be: (g, 0)),
    ),
    out_shape=jax.ShapeDtypeStruct((NR, H // 2), jnp.int32),
    compiler_params=pltpu.CompilerParams(
        dimension_semantics=("arbitrary",),
        vmem_limit_bytes=100 * 1024 * 1024,
    ),
)


_TOK_PER = S // NW
_MCH = 16


_MNCH = _TOK_PER // _MCH


@functools.partial(
    pl.kernel,
    mesh=_mesh,
    compiler_params=pltpu.CompilerParams(needs_layout_passes=False),
    out_type=jax.ShapeDtypeStruct((S, H), jnp.float32),
    scratch_types=[
        pltpu.VMEM((_TOK_PER,), jnp.int32),
        pltpu.VMEM((_TOK_PER,), jnp.int32),
    ] + [pltpu.VMEM((_MCH, H // 2), jnp.int32) for _ in range(4)]
      + [pltpu.VMEM((_MCH, H), jnp.float32) for _ in range(2)]
      + [pltpu.SemaphoreType.DMA for _ in range(6)],
)
def _merge(y_hbm, pos_hbm, out_hbm, p0_v, p1_v, *bufsem):
    y0 = bufsem[0:2]
    y1 = bufsem[2:4]
    ov = bufsem[4:6]
    sa = bufsem[6:8]
    sb = bufsem[8:10]
    so = bufsem[10:12]
    wid = lax.axis_index("s") * 2 + lax.axis_index("c")
    base = wid * _TOK_PER
    pltpu.sync_copy(pos_hbm.at[pl.ds(base, _TOK_PER)], p0_v)
    pltpu.sync_copy(pos_hbm.at[pl.ds(S + base, _TOK_PER)], p1_v)

    d0 = [None] * _MNCH
    d1 = [None] * _MNCH
    for c in range(2):
        d0[c] = pltpu.async_copy(
            y_hbm.at[p0_v.at[pl.ds(c * _MCH, _MCH)]], y0[c % 2], sa[c % 2])
        d1[c] = pltpu.async_copy(
            y_hbm.at[p1_v.at[pl.ds(c * _MCH, _MCH)]], y1[c % 2], sb[c % 2])
    for c in range(_MNCH):
        p = c % 2
        d0[c].wait()
        d1[c].wait()

        def add_body(i, _, y0r=y0[p], y1r=y1[p], ovr=ov[p]):
            r = i >> 5
            w = (i & 31) * LANES
            v0 = y0r[r, pl.ds(w, LANES)]
            v1 = y1r[r, pl.ds(w, LANES)]
            ev = (plsc.bitcast(v0 << 16, jnp.float32)
                  + plsc.bitcast(v1 << 16, jnp.float32))
            od_ = (plsc.bitcast(v0 & (-65536), jnp.float32)
                   + plsc.bitcast(v1 & (-65536), jnp.float32))
            ovr[r, pl.ds(w, LANES)] = ev
            ovr[r, pl.ds(H // 2 + w, LANES)] = od_
            return 0

        lax.fori_loop(0, _MCH * (H // (2 * LANES)), add_body, 0)
        od = pltpu.async_copy(
            ov[p], out_hbm.at[pl.ds(base + c * _MCH, _MCH)], so[p])
        od.wait()
        if c + 2 < _MNCH:
            d0[c + 2] = pltpu.async_copy(
                y_hbm.at[p0_v.at[pl.ds((c + 2) * _MCH, _MCH)]], y0[p], sa[p])
            d1[c + 2] = pltpu.async_copy(
                y_hbm.at[p1_v.at[pl.ds((c + 2) * _MCH, _MCH)]], y1[p], sb[p])


def kernel(hidden_states, gate_w, w1, w2, w3):
    b, s, h = hidden_states.shape
    x2d = hidden_states.reshape(s, h)
    e0, e1, wt0, wt1 = _router(x2d, gate_w)
    be, rt, rw, pos = _dispatch_meta(e0, e1, wt0, wt1)
    py = _mlp(be, rt.reshape(G, R, 1), x2d, w1, w3, w2, rw.reshape(NR, 1))
    out = _merge(py, pos)
    return out.reshape(b, s, h)


# speedup vs baseline: 1.8009x; 1.0088x over previous
import functools

import jax
import jax.numpy as jnp
from jax import lax
from jax.experimental import pallas as pl
from jax.experimental.pallas import tpu as pltpu
from jax.experimental.pallas import tpu_sc as plsc

E = 8
H = 1024
FF = 2048
S = 2048
NSLOT = 2 * S
R = 256
G = 24
NR = G * R
LANES = 16
NW = 32

_mesh = plsc.VectorSubcoreMesh(core_axis_name="c", subcore_axis_name="s")


def _router_body(x_ref, gw_ref, e0_ref, e1_ref, w0_ref, w1_ref):
    l = jnp.dot(x_ref[...], gw_ref[...], preferred_element_type=jnp.float32)
    idx = lax.broadcasted_iota(jnp.int32, l.shape, 1)
    m0 = jnp.max(l, axis=1, keepdims=True)
    e0 = jnp.min(jnp.where(l == m0, idx, E), axis=1, keepdims=True)
    l2 = jnp.where(idx == e0, -jnp.inf, l)
    m1 = jnp.max(l2, axis=1, keepdims=True)
    e1 = jnp.min(jnp.where(l2 == m1, idx, E), axis=1, keepdims=True)
    p0 = 1.0 / (1.0 + jnp.exp(m1 - m0))
    e0_ref[...] = e0.reshape(S)
    e1_ref[...] = e1.reshape(S)
    w0_ref[...] = p0.reshape(S)
    w1_ref[...] = 1.0 - p0.reshape(S)


_router = pl.pallas_call(
    _router_body,
    out_shape=[
        jax.ShapeDtypeStruct((S,), jnp.int32),
        jax.ShapeDtypeStruct((S,), jnp.int32),
        jax.ShapeDtypeStruct((S,), jnp.float32),
        jax.ShapeDtypeStruct((S,), jnp.float32),
    ],
)


@functools.partial(
    pl.kernel,
    mesh=_mesh,
    compiler_params=pltpu.CompilerParams(needs_layout_passes=False),
    out_type=[
        jax.ShapeDtypeStruct((32,), jnp.int32),
        jax.ShapeDtypeStruct((NR,), jnp.int32),
        jax.ShapeDtypeStruct((NR,), jnp.float32),
        jax.ShapeDtypeStruct((NSLOT,), jnp.int32),
    ],
    scratch_types=[
        pltpu.VMEM((NSLOT,), jnp.int32),
        pltpu.VMEM((NSLOT,), jnp.float32),
        pltpu.VMEM((NR,), jnp.int32),
        pltpu.VMEM((NR,), jnp.float32),
        pltpu.VMEM((NSLOT,), jnp.int32),
        pltpu.VMEM((32,), jnp.int32),
    ],
)
def _dispatch_meta(e0_hbm, e1_hbm, w0_hbm, w1_hbm,
                   be_hbm, rt_hbm, rw_hbm, pos_hbm,
                   sel_v, w_v, rt_v, rw_v, pos_v, be_v):
    wid = lax.axis_index("s") * 2 + lax.axis_index("c")

    @pl.when(wid == 0)
    def _():
        pltpu.sync_copy(e0_hbm, sel_v.at[pl.ds(0, S)])
        pltpu.sync_copy(e1_hbm, sel_v.at[pl.ds(S, S)])
        pltpu.sync_copy(w0_hbm, w_v.at[pl.ds(0, S)])
        pltpu.sync_copy(w1_hbm, w_v.at[pl.ds(S, S)])

        nvec = NSLOT // LANES

        def hist_body(i, acc):
            v = sel_v[pl.ds(i * LANES, LANES)]
            return tuple(acc[e] + (v == e).astype(jnp.int32) for e in range(E))

        acc = lax.fori_loop(
            0, nvec, hist_body,
            tuple(jnp.zeros((LANES,), jnp.int32) for _ in range(E)))
        tot = [jnp.sum(a) for a in acc]
        nb = [(t + (R - 1)) >> 8 for t in tot]
        bs = []
        run = jnp.int32(0)
        for e in range(E):
            bs.append(run)
            run = run + nb[e]
        tb = run

        def z_body(i, _):
            rt_v[pl.ds(i * LANES, LANES)] = jnp.zeros((LANES,), jnp.int32)
            rw_v[pl.ds(i * LANES, LANES)] = jnp.zeros((LANES,), jnp.float32)
            return 0

        lax.fori_loop(0, NR // LANES, z_body, 0)

        def p2_body(i, cur):
            v = sel_v[pl.ds(i * LANES, LANES)]
            w = w_v[pl.ds(i * LANES, LANES)]
            pos = jnp.zeros((LANES,), jnp.int32)
            ncur = []
            for e in range(E):
                m = v == e
                mi = m.astype(jnp.int32)
                csum = plsc.cumsum(mi)
                pos = jnp.where(m, cur[e] + csum - 1, pos)
                ncur.append(cur[e] + jnp.sum(mi))
            tok = (i * LANES + lax.iota(jnp.int32, 16)) & (S - 1)
            plsc.store_scatter(rt_v, [pos], tok)
            plsc.store_scatter(rw_v, [pos], w)
            pos_v[pl.ds(i * LANES, LANES)] = pos
            return tuple(ncur)

        lax.fori_loop(0, nvec, p2_body, tuple(bs[e] * R for e in range(E)))

        for j in range(2):
            g = lax.iota(jnp.int32, 16) + j * LANES
            ge = jnp.minimum(g, tb - 1)
            be = jnp.zeros((LANES,), jnp.int32)
            for e in range(1, E):
                be = be + (ge >= bs[e]).astype(jnp.int32)
            if j == 1:
                be = jnp.where(lax.iota(jnp.int32, 16) == 15, tb, be)
            be_v[pl.ds(j * LANES, LANES)] = be

        pltpu.sync_copy(be_v, be_hbm)
        pltpu.sync_copy(rt_v, rt_hbm)
        pltpu.sync_copy(rw_v, rw_hbm)
        pltpu.sync_copy(pos_v, pos_hbm)


def _mlp_body(be_ref, rt_ref, x_ref, w1_ref, w3_ref, w2_ref, rw_ref, y_ref):
    g = pl.program_id(0)

    @pl.when(g < be_ref[31])
    def _():
        tok = rt_ref[0]
        ids = lax.broadcasted_iota(jnp.int32, (R, S), 1)
        onehot = (ids == tok).astype(jnp.float32)
        x = jnp.dot(onehot, x_ref[...], preferred_element_type=jnp.float32)
        a = jnp.dot(x, w1_ref[0], preferred_element_type=jnp.float32)
        u = jnp.dot(x, w3_ref[0], preferred_element_type=jnp.float32)
        h = (a / (1.0 + jnp.exp(-a))) * u
        y = jnp.dot(h, w2_ref[0], preferred_element_type=jnp.float32)
        y = y * rw_ref[...]
        u1 = jax.lax.bitcast_convert_type(y[:, :H // 2], jnp.int32)
        u2 = jax.lax.bitcast_convert_type(y[:, H // 2:], jnp.int32)
        lo = ((u1 + 0x8000) >> 16) & 0xFFFF
        hi = (u2 + 0x8000) & (-65536)
        y_ref[...] = lo | hi


_mlp = pl.pallas_call(
    _mlp_body,
    grid_spec=pltpu.PrefetchScalarGridSpec(
        num_scalar_prefetch=1,
        grid=(G,),
        in_specs=[
            pl.BlockSpec((1, R, 1), lambda g, be: (g, 0, 0)),
            pl.BlockSpec((S, H), lambda g, be: (0, 0)),
            pl.BlockSpec((1, H, FF), lambda g, be: (be[g], 0, 0)),
            pl.BlockSpec((1, H, FF), lambda g, be: (be[g], 0, 0)),
            pl.BlockSpec((1, FF, H), lambda g, be: (be[g], 0, 0)),
            pl.BlockSpec((R, 1), lambda g, be: (g, 0)),
        ],
        out_specs=pl.BlockSpec((R, H // 2), lambda g, be: (g, 0)),
    ),
    out_shape=jax.ShapeDtypeStruct((NR, H // 2), jnp.int32),
    compiler_params=pltpu.CompilerParams(
        dimension_semantics=("arbitrary",),
        vmem_limit_bytes=100 * 1024 * 1024,
    ),
)


_TOK_PER = S // NW
_MCH = 16


_MNCH = _TOK_PER // _MCH


@functools.partial(
    pl.kernel,
    mesh=_mesh,
    compiler_params=pltpu.CompilerParams(needs_layout_passes=False),
    out_type=jax.ShapeDtypeStruct((S, H), jnp.float32),
    scratch_types=[
        pltpu.VMEM((_TOK_PER,), jnp.int32),
        pltpu.VMEM((_TOK_PER,), jnp.int32),
    ] + [pltpu.VMEM((_MCH, H // 2), jnp.int32) for _ in range(4)]
      + [pltpu.VMEM((_MCH, H), jnp.float32) for _ in range(2)]
      + [pltpu.SemaphoreType.DMA for _ in range(6)],
)
def _merge(y_hbm, pos_hbm, out_hbm, p0_v, p1_v, *bufsem):
    y0 = bufsem[0:2]
    y1 = bufsem[2:4]
    ov = bufsem[4:6]
    sa = bufsem[6:8]
    sb = bufsem[8:10]
    so = bufsem[10:12]
    wid = lax.axis_index("s") * 2 + lax.axis_index("c")
    base = wid * _TOK_PER
    pltpu.sync_copy(pos_hbm.at[pl.ds(base, _TOK_PER)], p0_v)
    pltpu.sync_copy(pos_hbm.at[pl.ds(S + base, _TOK_PER)], p1_v)

    d0 = [None] * _MNCH
    d1 = [None] * _MNCH
    o_desc = [None, None]
    for c in range(2):
        d0[c] = pltpu.async_copy(
            y_hbm.at[p0_v.at[pl.ds(c * _MCH, _MCH)]], y0[c % 2], sa[c % 2])
        d1[c] = pltpu.async_copy(
            y_hbm.at[p1_v.at[pl.ds(c * _MCH, _MCH)]], y1[c % 2], sb[c % 2])
    for c in range(_MNCH):
        p = c % 2
        d0[c].wait()
        d1[c].wait()
        if o_desc[p] is not None:
            o_desc[p].wait()

        def add_body(i, _, y0r=y0[p], y1r=y1[p], ovr=ov[p]):
            r = i >> 5
            w = (i & 31) * LANES
            v0 = y0r[r, pl.ds(w, LANES)]
            v1 = y1r[r, pl.ds(w, LANES)]
            ev = (plsc.bitcast(v0 << 16, jnp.float32)
                  + plsc.bitcast(v1 << 16, jnp.float32))
            od_ = (plsc.bitcast(v0 & (-65536), jnp.float32)
                   + plsc.bitcast(v1 & (-65536), jnp.float32))
            ovr[r, pl.ds(w, LANES)] = ev
            ovr[r, pl.ds(H // 2 + w, LANES)] = od_
            return 0

        lax.fori_loop(0, _MCH * (H // (2 * LANES)), add_body, 0)
        o_desc[p] = pltpu.async_copy(
            ov[p], out_hbm.at[pl.ds(base + c * _MCH, _MCH)], so[p])
        if c + 2 < _MNCH:
            d0[c + 2] = pltpu.async_copy(
                y_hbm.at[p0_v.at[pl.ds((c + 2) * _MCH, _MCH)]], y0[p], sa[p])
            d1[c + 2] = pltpu.async_copy(
                y_hbm.at[p1_v.at[pl.ds((c + 2) * _MCH, _MCH)]], y1[p], sb[p])
    o_desc[0].wait()
    o_desc[1].wait()


def kernel(hidden_states, gate_w, w1, w2, w3):
    b, s, h = hidden_states.shape
    x2d = hidden_states.reshape(s, h)
    e0, e1, wt0, wt1 = _router(x2d, gate_w)
    be, rt, rw, pos = _dispatch_meta(e0, e1, wt0, wt1)
    py = _mlp(be, rt.reshape(G, R, 1), x2d, w1, w3, w2, rw.reshape(NR, 1))
    out = _merge(py, pos)
    return out.reshape(b, s, h)
